# Initial kernel scaffold; baseline (speedup 1.0000x reference)
import jax
import jax.numpy as jnp
from jax.experimental import pallas as pl


def kernel(x, emb, W1, b1, W2, b2):
    raise NotImplementedError("write your pallas kernel here")


# speedup vs baseline: 1.7453x; 1.7453x over previous
import jax
import jax.numpy as jnp
from jax import lax
from jax.experimental import pallas as pl
from jax.experimental.pallas import tpu as pltpu
from jax.experimental.pallas import tpu_sc as plsc

L = 16
DP = 64


def _sc_pool(x3, embp, nc, ns, rpw, nchunk, ch):
    nw = nc * ns
    b = nw * rpw
    seq = x3.shape[2] // ch
    chi = ch * seq
    nwin = DP // L

    def body(x_hbm, emb_hbm, osum_hbm, ocnt_hbm,
             x_v, g0, g1, osum_v, ocnt_v, sem0, sem1):
        cid = lax.axis_index("c")
        sid = lax.axis_index("s")
        wid = sid * nc + cid
        row0 = wid * rpw

        pltpu.sync_copy(x_hbm.at[wid], x_v)

        gbufs = (g0, g1)
        sems = (sem0, sem1)

        pltpu.make_async_copy(emb_hbm.at[x_v.at[0]], g0, sem0).start()
        pltpu.make_async_copy(emb_hbm.at[x_v.at[1]], g1, sem1).start()

        def compute_chunk(c, gb):
            for r in range(ch):
                def lbody(l, carry):
                    accs = list(carry)
                    row = r * seq + l
                    for w in range(nwin):
                        v = gb[row, pl.ds(w * L, L)]
                        accs[w] = accs[w] + v
                        accs[nwin + w] = accs[nwin + w] + jnp.where(
                            v == 0.0, 0.0, 1.0)
                    return tuple(accs)

                zero = jnp.zeros((L,), jnp.float32)
                init = tuple(zero for _ in range(2 * nwin))
                accs = lax.fori_loop(0, seq, lbody, init)
                orow = c * ch + r
                for w in range(nwin):
                    osum_v[orow, pl.ds(w * L, L)] = accs[w]
                    ocnt_v[orow, pl.ds(w * L, L)] = accs[nwin + w]

        def cbody(cc, _):
            for buf in range(2):
                c = 2 * cc + buf
                gb = gbufs[buf]
                sem = sems[buf]
                pltpu.make_async_copy(emb_hbm.at[x_v.at[0]], gb, sem).wait()
                compute_chunk(c, gb)

                @pl.when(c + 2 < nchunk)
                def _():
                    pltpu.make_async_copy(
                        emb_hbm.at[x_v.at[c + 2]], gb, sem).start()
            return 0

        lax.fori_loop(0, nchunk // 2, cbody, 0)

        pltpu.sync_copy(osum_v, osum_hbm.at[pl.ds(row0, rpw)])
        pltpu.sync_copy(ocnt_v, ocnt_hbm.at[pl.ds(row0, rpw)])

    mesh = plsc.VectorSubcoreMesh(core_axis_name="c", subcore_axis_name="s",
                                  num_cores=nc, num_subcores=ns)
    fn = pl.kernel(
        body,
        out_type=(jax.ShapeDtypeStruct((b, DP), jnp.float32),
                  jax.ShapeDtypeStruct((b, DP), jnp.float32)),
        mesh=mesh,
        compiler_params=pltpu.CompilerParams(use_tc_tiling_on_sc=False),
        scratch_types=[
            pltpu.VMEM((nchunk, chi), jnp.int32),
            pltpu.VMEM((chi, DP), jnp.float32),
            pltpu.VMEM((chi, DP), jnp.float32),
            pltpu.VMEM((rpw, DP), jnp.float32),
            pltpu.VMEM((rpw, DP), jnp.float32),
            pltpu.SemaphoreType.DMA,
            pltpu.SemaphoreType.DMA,
        ],
    )
    return fn(x3, embp)


def _mlp_body(hs_ref, hc_ref, w1_ref, b1_ref, w2_ref, b2_ref, o_ref):
    d = w1_ref.shape[1]
    h0 = hs_ref[:, :d] / hc_ref[:, :d]
    h1 = lax.dot_general(h0, w1_ref[...], (((1,), (1,)), ((), ())),
                         preferred_element_type=jnp.float32)
    h1 = jnp.maximum(h1 + b1_ref[...], 0.0)
    h2 = lax.dot_general(h1, w2_ref[...], (((1,), (1,)), ((), ())),
                         preferred_element_type=jnp.float32)
    o_ref[...] = h2 + b2_ref[...]


def _mlp(hsum, hcnt, w1, b1, w2, b2, blk=2048):
    b = hsum.shape[0]
    d = w1.shape[1]
    h1d = w1.shape[0]
    ncls = w2.shape[0]
    return pl.pallas_call(
        _mlp_body,
        grid=(b // blk,),
        in_specs=[
            pl.BlockSpec((blk, DP), lambda i: (i, 0)),
            pl.BlockSpec((blk, DP), lambda i: (i, 0)),
            pl.BlockSpec((h1d, d), lambda i: (0, 0)),
            pl.BlockSpec((1, h1d), lambda i: (0, 0)),
            pl.BlockSpec((ncls, h1d), lambda i: (0, 0)),
            pl.BlockSpec((1, ncls), lambda i: (0, 0)),
        ],
        out_specs=pl.BlockSpec((blk, ncls), lambda i: (i, 0)),
        out_shape=jax.ShapeDtypeStruct((b, ncls), jnp.float32),
    )(hsum, hcnt, w1, b1.reshape(1, h1d), w2, b2.reshape(1, ncls))


def kernel(x, emb, W1, b1, W2, b2):
    b, seq = x.shape
    nc, ns = 2, 16
    nw = nc * ns
    rpw = b // nw
    ch = 2
    nchunk = rpw // ch
    x3 = x.reshape(nw, nchunk, ch * seq)
    embp = jnp.pad(emb, ((0, 0), (0, DP - emb.shape[1])))
    hsum, hcnt = _sc_pool(x3, embp, nc, ns, rpw, nchunk, ch)
    return _mlp(hsum, hcnt, W1, b1, W2, b2)


# speedup vs baseline: 2.1673x; 1.2418x over previous
import jax
import jax.numpy as jnp
from jax import lax
from jax.experimental import pallas as pl
from jax.experimental.pallas import tpu as pltpu
from jax.experimental.pallas import tpu_sc as plsc

L = 16
DP = 128
DW = 64


def _sc_pool(x3, embp, nc, ns, rpw, nchunk, ch):
    nw = nc * ns
    b = nw * rpw
    seq = x3.shape[2] // ch
    chi = ch * seq
    nwin = DW // L

    def body(x_hbm, emb_hbm, out_hbm, x_v, g0, g1, o_v, sem0, sem1):
        cid = lax.axis_index("c")
        sid = lax.axis_index("s")
        wid = sid * nc + cid
        row0 = wid * rpw

        pltpu.sync_copy(x_hbm.at[wid], x_v)

        gbufs = (g0, g1)
        sems = (sem0, sem1)

        pltpu.make_async_copy(emb_hbm.at[x_v.at[0]], g0, sem0).start()
        pltpu.make_async_copy(emb_hbm.at[x_v.at[1]], g1, sem1).start()

        def compute_chunk(c, gb):
            for r in range(ch):
                def lbody(l, carry):
                    accs = list(carry)
                    row = r * seq + l
                    for w in range(nwin):
                        v = gb[row, pl.ds(w * L, L)]
                        accs[w] = accs[w] + v
                        accs[nwin + w] = accs[nwin + w] + jnp.where(
                            v == 0.0, 0.0, 1.0)
                    return tuple(accs)

                zero = jnp.zeros((L,), jnp.float32)
                init = tuple(zero for _ in range(2 * nwin))
                accs = lax.fori_loop(0, seq, lbody, init)
                orow = c * ch + r
                for w in range(2 * nwin):
                    o_v[orow, pl.ds(w * L, L)] = accs[w]

        def cbody(cc, _):
            for buf in range(2):
                c = 2 * cc + buf
                gb = gbufs[buf]
                sem = sems[buf]
                pltpu.make_async_copy(emb_hbm.at[x_v.at[0]], gb, sem).wait()
                compute_chunk(c, gb)

                @pl.when(c + 2 < nchunk)
                def _():
                    pltpu.make_async_copy(
                        emb_hbm.at[x_v.at[c + 2]], gb, sem).start()
            return 0

        lax.fori_loop(0, nchunk // 2, cbody, 0)

        pltpu.sync_copy(o_v, out_hbm.at[pl.ds(row0, rpw)])

    mesh = plsc.VectorSubcoreMesh(core_axis_name="c", subcore_axis_name="s",
                                  num_cores=nc, num_subcores=ns)
    fn = pl.kernel(
        body,
        out_type=jax.ShapeDtypeStruct((b, DP), jnp.float32),
        mesh=mesh,
        compiler_params=pltpu.CompilerParams(use_tc_tiling_on_sc=True),
        scratch_types=[
            pltpu.VMEM((nchunk, chi), jnp.int32),
            pltpu.VMEM((chi, DP), jnp.float32),
            pltpu.VMEM((chi, DP), jnp.float32),
            pltpu.VMEM((rpw, DP), jnp.float32),
            pltpu.SemaphoreType.DMA,
            pltpu.SemaphoreType.DMA,
        ],
    )
    return fn(x3, embp)


def _mlp_body(sc_ref, w1_ref, b1_ref, w2_ref, b2_ref, o_ref):
    d = w1_ref.shape[1]
    h0 = sc_ref[:, :d] / sc_ref[:, DW:DW + d]
    h1 = lax.dot_general(h0, w1_ref[...], (((1,), (1,)), ((), ())),
                         preferred_element_type=jnp.float32)
    h1 = jnp.maximum(h1 + b1_ref[...], 0.0)
    h2 = lax.dot_general(h1, w2_ref[...], (((1,), (1,)), ((), ())),
                         preferred_element_type=jnp.float32)
    o_ref[...] = h2 + b2_ref[...]


def _mlp(sc_out, w1, b1, w2, b2, blk=2048):
    b = sc_out.shape[0]
    d = w1.shape[1]
    h1d = w1.shape[0]
    ncls = w2.shape[0]
    return pl.pallas_call(
        _mlp_body,
        grid=(b // blk,),
        in_specs=[
            pl.BlockSpec((blk, DP), lambda i: (i, 0)),
            pl.BlockSpec((h1d, d), lambda i: (0, 0)),
            pl.BlockSpec((1, h1d), lambda i: (0, 0)),
            pl.BlockSpec((ncls, h1d), lambda i: (0, 0)),
            pl.BlockSpec((1, ncls), lambda i: (0, 0)),
        ],
        out_specs=pl.BlockSpec((blk, ncls), lambda i: (i, 0)),
        out_shape=jax.ShapeDtypeStruct((b, ncls), jnp.float32),
    )(sc_out, w1, b1.reshape(1, h1d), w2, b2.reshape(1, ncls))


def kernel(x, emb, W1, b1, W2, b2):
    b, seq = x.shape
    nc, ns = 2, 16
    nw = nc * ns
    rpw = b // nw
    ch = 2
    nchunk = rpw // ch
    x3 = x.reshape(nw, nchunk, ch * seq)
    embp = jnp.pad(emb, ((0, 0), (0, DP - emb.shape[1])))
    sc_out = _sc_pool(x3, embp, nc, ns, rpw, nchunk, ch)
    return _mlp(sc_out, W1, b1, W2, b2)


# speedup vs baseline: 3.2709x; 1.5092x over previous
import jax
import jax.numpy as jnp
from jax import lax
from jax.experimental import pallas as pl
from jax.experimental.pallas import tpu as pltpu
from jax.experimental.pallas import tpu_sc as plsc

L = 16
TW = 128
GW = 64
OW = 128


def _pad_body(e_ref, o_ref):
    d = e_ref.shape[1]
    o_ref[:, :d] = e_ref[...]
    o_ref[:, d:] = jnp.zeros_like(o_ref[:, d:])


def _pad_table(emb, blk=8000):
    v, d = emb.shape
    return pl.pallas_call(
        _pad_body,
        grid=(v // blk,),
        in_specs=[pl.BlockSpec((blk, d), lambda i: (i, 0))],
        out_specs=pl.BlockSpec((blk, TW), lambda i: (i, 0)),
        out_shape=jax.ShapeDtypeStruct((v, TW), jnp.float32),
    )(emb)


def _sc_pool(x3, embp, nc, ns, rpw, nchunk, ch):
    nw = nc * ns
    b = nw * rpw
    seq = x3.shape[2] // ch
    chi = ch * seq
    nwin = GW // L

    def body(x_hbm, emb_hbm, out_hbm, x_v, g0, g1, o_v, sem0, sem1):
        cid = lax.axis_index("c")
        sid = lax.axis_index("s")
        wid = sid * nc + cid
        row0 = wid * rpw

        pltpu.sync_copy(x_hbm.at[wid], x_v)

        gbufs = (g0, g1)
        sems = (sem0, sem1)

        pltpu.make_async_copy(emb_hbm.at[x_v.at[0]], g0, sem0).start()
        pltpu.make_async_copy(emb_hbm.at[x_v.at[1]], g1, sem1).start()

        def compute_chunk(c, gb):
            for r in range(ch):
                def lbody(l, carry):
                    accs = list(carry)
                    row = r * seq + l
                    for w in range(nwin):
                        v = gb[row, pl.ds(w * L, L)]
                        accs[w] = accs[w] + v
                        accs[nwin + w] = accs[nwin + w] + jnp.where(
                            v == 0.0, 0.0, 1.0)
                    return tuple(accs)

                zero = jnp.zeros((L,), jnp.float32)
                init = tuple(zero for _ in range(2 * nwin))
                accs = lax.fori_loop(0, seq, lbody, init)
                orow = c * ch + r
                for w in range(2 * nwin):
                    o_v[orow, pl.ds(w * L, L)] = accs[w]

        def cbody(cc, _):
            for buf in range(2):
                c = 2 * cc + buf
                gb = gbufs[buf]
                sem = sems[buf]
                pltpu.make_async_copy(emb_hbm.at[x_v.at[0]], gb, sem).wait()
                compute_chunk(c, gb)

                @pl.when(c + 2 < nchunk)
                def _():
                    pltpu.make_async_copy(
                        emb_hbm.at[x_v.at[c + 2]], gb, sem).start()
            return 0

        lax.fori_loop(0, nchunk // 2, cbody, 0)

        pltpu.sync_copy(o_v, out_hbm.at[pl.ds(row0, rpw)])

    mesh = plsc.VectorSubcoreMesh(core_axis_name="c", subcore_axis_name="s",
                                  num_cores=nc, num_subcores=ns)
    fn = pl.kernel(
        body,
        out_type=jax.ShapeDtypeStruct((b, OW), jnp.float32),
        mesh=mesh,
        compiler_params=pltpu.CompilerParams(use_tc_tiling_on_sc=True),
        scratch_types=[
            pltpu.VMEM((nchunk, chi), jnp.int32),
            pltpu.VMEM((chi, TW), jnp.float32),
            pltpu.VMEM((chi, TW), jnp.float32),
            pltpu.VMEM((rpw, OW), jnp.float32),
            pltpu.SemaphoreType.DMA,
            pltpu.SemaphoreType.DMA,
        ],
    )
    return fn(x3, embp)


def _mlp_body(sc_ref, w1_ref, b1_ref, w2_ref, b2_ref, o_ref):
    d = w1_ref.shape[1]
    h0 = sc_ref[:, :d] / sc_ref[:, GW:GW + d]
    h1 = lax.dot_general(h0, w1_ref[...], (((1,), (1,)), ((), ())),
                         preferred_element_type=jnp.float32)
    h1 = jnp.maximum(h1 + b1_ref[...], 0.0)
    h2 = lax.dot_general(h1, w2_ref[...], (((1,), (1,)), ((), ())),
                         preferred_element_type=jnp.float32)
    o_ref[...] = h2 + b2_ref[...]


def _mlp(sc_out, w1, b1, w2, b2, blk=2048):
    b = sc_out.shape[0]
    d = w1.shape[1]
    h1d = w1.shape[0]
    ncls = w2.shape[0]
    return pl.pallas_call(
        _mlp_body,
        grid=(b // blk,),
        in_specs=[
            pl.BlockSpec((blk, OW), lambda i: (i, 0)),
            pl.BlockSpec((h1d, d), lambda i: (0, 0)),
            pl.BlockSpec((1, h1d), lambda i: (0, 0)),
            pl.BlockSpec((ncls, h1d), lambda i: (0, 0)),
            pl.BlockSpec((1, ncls), lambda i: (0, 0)),
        ],
        out_specs=pl.BlockSpec((blk, ncls), lambda i: (i, 0)),
        out_shape=jax.ShapeDtypeStruct((b, ncls), jnp.float32),
    )(sc_out, w1, b1.reshape(1, h1d), w2, b2.reshape(1, ncls))


def kernel(x, emb, W1, b1, W2, b2):
    b, seq = x.shape
    nc, ns = 2, 16
    nw = nc * ns
    rpw = b // nw
    ch = 2
    nchunk = rpw // ch
    x3 = x.reshape(nw, nchunk, ch * seq)
    embp = _pad_table(emb)
    sc_out = _sc_pool(x3, embp, nc, ns, rpw, nchunk, ch)
    return _mlp(sc_out, W1, b1, W2, b2)


# speedup vs baseline: 3.2880x; 1.0052x over previous
import jax
import jax.numpy as jnp
from jax import lax
from jax.experimental import pallas as pl
from jax.experimental.pallas import tpu as pltpu
from jax.experimental.pallas import tpu_sc as plsc

L = 16
TW = 128
GW = 64
OW = 128


def _pad_body(e_ref, o_ref):
    d = e_ref.shape[1]
    o_ref[:, :d] = e_ref[...]
    o_ref[:, d:] = jnp.zeros_like(o_ref[:, d:])


def _pad_table(emb, blk=25000):
    v, d = emb.shape
    return pl.pallas_call(
        _pad_body,
        grid=(v // blk,),
        in_specs=[pl.BlockSpec((blk, d), lambda i: (i, 0))],
        out_specs=pl.BlockSpec((blk, TW), lambda i: (i, 0)),
        out_shape=jax.ShapeDtypeStruct((v, TW), jnp.float32),
    )(emb)


def _sc_pool(x3, embp, nc, ns, rpw, nchunk, ch):
    nw = nc * ns
    b = nw * rpw
    seq = x3.shape[2] // ch
    chi = ch * seq
    nwin = GW // L

    def body(x_hbm, emb_hbm, out_hbm, x_v, g0, g1, o_v, sem0, sem1):
        cid = lax.axis_index("c")
        sid = lax.axis_index("s")
        wid = sid * nc + cid
        row0 = wid * rpw

        pltpu.sync_copy(x_hbm.at[wid], x_v)

        gbufs = (g0, g1)
        sems = (sem0, sem1)

        pltpu.make_async_copy(emb_hbm.at[x_v.at[0]], g0, sem0).start()
        pltpu.make_async_copy(emb_hbm.at[x_v.at[1]], g1, sem1).start()

        def compute_chunk(c, gb):
            for r in range(ch):
                def lbody(l, carry):
                    accs = list(carry)
                    row = r * seq + l
                    for w in range(nwin):
                        v = gb[row, pl.ds(w * L, L)]
                        accs[w] = accs[w] + v
                        accs[nwin + w] = accs[nwin + w] + jnp.where(
                            v == 0.0, 0.0, 1.0)
                    return tuple(accs)

                zero = jnp.zeros((L,), jnp.float32)
                init = tuple(zero for _ in range(2 * nwin))
                accs = lax.fori_loop(0, seq, lbody, init)
                orow = c * ch + r
                for w in range(2 * nwin):
                    o_v[orow, pl.ds(w * L, L)] = accs[w]

        def cbody(cc, _):
            for buf in range(2):
                c = 2 * cc + buf
                gb = gbufs[buf]
                sem = sems[buf]
                pltpu.make_async_copy(emb_hbm.at[x_v.at[0]], gb, sem).wait()
                compute_chunk(c, gb)

                @pl.when(c + 2 < nchunk)
                def _():
                    pltpu.make_async_copy(
                        emb_hbm.at[x_v.at[c + 2]], gb, sem).start()
            return 0

        lax.fori_loop(0, nchunk // 2, cbody, 0)

        pltpu.sync_copy(o_v, out_hbm.at[pl.ds(row0, rpw)])

    mesh = plsc.VectorSubcoreMesh(core_axis_name="c", subcore_axis_name="s",
                                  num_cores=nc, num_subcores=ns)
    fn = pl.kernel(
        body,
        out_type=jax.ShapeDtypeStruct((b, OW), jnp.float32),
        mesh=mesh,
        compiler_params=pltpu.CompilerParams(use_tc_tiling_on_sc=True),
        scratch_types=[
            pltpu.VMEM((nchunk, chi), jnp.int32),
            pltpu.VMEM((chi, TW), jnp.float32),
            pltpu.VMEM((chi, TW), jnp.float32),
            pltpu.VMEM((rpw, OW), jnp.float32),
            pltpu.SemaphoreType.DMA,
            pltpu.SemaphoreType.DMA,
        ],
    )
    return fn(x3, embp)


def _mlp_body(sc_ref, w1_ref, b1_ref, w2_ref, b2_ref, o_ref):
    d = w1_ref.shape[1]
    h0 = sc_ref[:, :d] / sc_ref[:, GW:GW + d]
    h1 = lax.dot_general(h0, w1_ref[...], (((1,), (1,)), ((), ())),
                         preferred_element_type=jnp.float32)
    h1 = jnp.maximum(h1 + b1_ref[...], 0.0)
    h2 = lax.dot_general(h1, w2_ref[...], (((1,), (1,)), ((), ())),
                         preferred_element_type=jnp.float32)
    o_ref[...] = h2 + b2_ref[...]


def _mlp(sc_out, w1, b1, w2, b2, blk=4096):
    b = sc_out.shape[0]
    d = w1.shape[1]
    h1d = w1.shape[0]
    ncls = w2.shape[0]
    return pl.pallas_call(
        _mlp_body,
        grid=(b // blk,),
        in_specs=[
            pl.BlockSpec((blk, OW), lambda i: (i, 0)),
            pl.BlockSpec((h1d, d), lambda i: (0, 0)),
            pl.BlockSpec((1, h1d), lambda i: (0, 0)),
            pl.BlockSpec((ncls, h1d), lambda i: (0, 0)),
            pl.BlockSpec((1, ncls), lambda i: (0, 0)),
        ],
        out_specs=pl.BlockSpec((blk, ncls), lambda i: (i, 0)),
        out_shape=jax.ShapeDtypeStruct((b, ncls), jnp.float32),
    )(sc_out, w1, b1.reshape(1, h1d), w2, b2.reshape(1, ncls))


def kernel(x, emb, W1, b1, W2, b2):
    b, seq = x.shape
    nc, ns = 2, 16
    nw = nc * ns
    rpw = b // nw
    ch = 2
    nchunk = rpw // ch
    x3 = x.reshape(nw, nchunk, ch * seq)
    embp = _pad_table(emb)
    sc_out = _sc_pool(x3, embp, nc, ns, rpw, nchunk, ch)
    return _mlp(sc_out, W1, b1, W2, b2)
